# baseline (speedup 1.0000x reference)
import jax
import jax.numpy as jnp
from jax.experimental import pallas as pl
from jax.experimental.pallas import tpu as pltpu
from jax.experimental.pallas import tpu_sc as plsc

_N1 = 10000
_N2 = 10000
_D = 64
_K = 3
_N1P = 10240
_N2P = 10240
_BQ = 256
_BC = 400
_PAD_COORD = 1.0e6
_BIG = 3.0e38

_DP = 128
_NW = 32
_RPW = _N2P * _K // _NW
_CHUNK = 120
_NCH = _RPW // _CHUNK


def _knn_body(q_ref, cT_ref, flowT_ref, idx_ref):
    q = q_ref[...]
    srcT = cT_ref[...] + flowT_ref[...]

    sx = srcT[0:1, :]
    sy = srcT[1:2, :]
    sz = srcT[2:3, :]
    src_sq = (sx * sx + sy * sy) + sz * sz

    qx = q[:, 0:1]
    qy = q[:, 1:2]
    qz = q[:, 2:3]
    q_sq = (qx * qx + qy * qy) + qz * qz

    dot = jnp.dot(q, srcT, preferred_element_type=jnp.float32)
    d2 = (q_sq + src_sq) - 2.0 * dot

    lane = jax.lax.broadcasted_iota(jnp.int32, (_BQ, _N1P), 1)

    idxs = []
    for _ in range(_K):
        idx = jnp.argmin(d2, axis=1)[:, None].astype(jnp.int32)
        d2 = jnp.where(lane == idx, _BIG, d2)
        idxs.append(idx)

    idx_ref[...] = jnp.concatenate(idxs, axis=1)


def _gather_body(table_hbm, idx_hbm, out_hbm, idx_v, rows_v, sem):
    wid = jax.lax.axis_index("s") * 2 + jax.lax.axis_index("c")
    base = wid * _RPW
    for c in range(_NCH):
        off = base + c * _CHUNK
        pltpu.sync_copy(idx_hbm.at[pl.ds(off, _CHUNK)], idx_v)
        pltpu.async_copy(table_hbm.at[idx_v], rows_v, sem).wait()
        pltpu.sync_copy(rows_v, out_hbm.at[pl.ds(off, _CHUNK)])


def _combine_body(g_ref, q_ref, c_ref, flow_ref, out_ref, warp_ref):
    warp_ref[...] = c_ref[...] + flow_ref[...]
    g = g_ref[...]
    q = q_ref[...]
    qx = q[:, 0:1]
    qy = q[:, 1:2]
    qz = q[:, 2:3]
    rs = []
    for k in range(_K):
        b = k * _DP + _D
        dx = (g[:, b:b + 1] + g[:, b + 3:b + 4]) - qx
        dy = (g[:, b + 1:b + 2] + g[:, b + 4:b + 5]) - qy
        dz = (g[:, b + 2:b + 3] + g[:, b + 5:b + 6]) - qz
        dist = jnp.sqrt((dx * dx + dy * dy) + dz * dz)
        rs.append(1.0 / jnp.maximum(dist, 1e-10))
    norm = (rs[0] + rs[1]) + rs[2]
    out_ref[...] = (g[:, 0:_D] * (rs[0] / norm)
                    + g[:, _DP:_DP + _D] * (rs[1] / norm)) \
        + g[:, 2 * _DP:2 * _DP + _D] * (rs[2] / norm)


def kernel(f_ref_C, f_ref_F, f_cur_C, point_flow):
    cT = jnp.pad(f_ref_C, ((0, _N1P - _N1), (0, 0)),
                 constant_values=_PAD_COORD).T
    flowT = jnp.pad(point_flow, ((0, _N1P - _N1), (0, 0))).T
    q_pad = jnp.pad(f_cur_C, ((0, _N2P - _N2), (0, 0)))
    table = jnp.pad(jnp.concatenate([f_ref_F, f_ref_C, point_flow], axis=1),
                    ((0, 0), (0, _DP - _D - 6)))

    knn_idx = pl.pallas_call(
        _knn_body,
        grid=(_N2P // _BQ,),
        in_specs=[
            pl.BlockSpec((_BQ, 3), lambda i: (i, 0)),
            pl.BlockSpec((3, _N1P), lambda i: (0, 0)),
            pl.BlockSpec((3, _N1P), lambda i: (0, 0)),
        ],
        out_specs=pl.BlockSpec((_BQ, _K), lambda i: (i, 0)),
        out_shape=jax.ShapeDtypeStruct((_N2P, _K), jnp.int32),
    )(q_pad, cT, flowT)

    idx_flat = knn_idx.reshape(_N2P * _K)
    mesh = plsc.VectorSubcoreMesh(core_axis_name="c", subcore_axis_name="s")
    grouped = pl.kernel(
        _gather_body,
        mesh=mesh,
        out_type=jax.ShapeDtypeStruct((_N2P * _K, _DP), jnp.float32),
        scratch_types=[
            pltpu.VMEM((_CHUNK,), jnp.int32),
            pltpu.VMEM((_CHUNK, _DP), jnp.float32),
            pltpu.SemaphoreType.DMA,
        ],
    )(table, idx_flat)

    out, f_ref_warp = pl.pallas_call(
        _combine_body,
        grid=(_N2 // _BC,),
        in_specs=[
            pl.BlockSpec((_BC, _K * _DP), lambda i: (i, 0)),
            pl.BlockSpec((_BC, 3), lambda i: (i, 0)),
            pl.BlockSpec((_BC, 3), lambda i: (i, 0)),
            pl.BlockSpec((_BC, 3), lambda i: (i, 0)),
        ],
        out_specs=[
            pl.BlockSpec((_BC, _D), lambda i: (i, 0)),
            pl.BlockSpec((_BC, 3), lambda i: (i, 0)),
        ],
        out_shape=[
            jax.ShapeDtypeStruct((_N2, _D), jnp.float32),
            jax.ShapeDtypeStruct((_N1, 3), jnp.float32),
        ],
    )(grouped.reshape(_N2P, _K * _DP), f_cur_C, f_ref_C, point_flow)

    return out, f_ref_warp


# speedup vs baseline: 1.0081x; 1.0081x over previous
import jax
import jax.numpy as jnp
from jax.experimental import pallas as pl
from jax.experimental.pallas import tpu as pltpu
from jax.experimental.pallas import tpu_sc as plsc

_N1 = 10000
_N2 = 10000
_D = 64
_K = 3
_N1P = 10240
_N2P = 10240
_BQ = 256
_BC = 512
_PAD_COORD = 1.0e6
_BIG = 3.0e38

_DP = 128
_NW = 32
_RPW = _N2P * _K // _NW
_CHUNK = 120
_NCH = _RPW // _CHUNK


def _warp_body(c_ref, flow_ref, out_ref):
    out_ref[...] = c_ref[...] + flow_ref[...]


def _knn_body(q_ref, srcT_ref, idx_ref):
    q = q_ref[...]
    srcT = srcT_ref[...]

    sx = srcT[0:1, :]
    sy = srcT[1:2, :]
    sz = srcT[2:3, :]
    src_sq = (sx * sx + sy * sy) + sz * sz

    qx = q[:, 0:1]
    qy = q[:, 1:2]
    qz = q[:, 2:3]
    q_sq = (qx * qx + qy * qy) + qz * qz

    dot = jnp.dot(q, srcT, preferred_element_type=jnp.float32)
    d2 = (q_sq + src_sq) - 2.0 * dot

    lane = jax.lax.broadcasted_iota(jnp.int32, (_BQ, _N1P), 1)

    idxs = []
    for _ in range(_K):
        idx = jnp.argmin(d2, axis=1)[:, None].astype(jnp.int32)
        d2 = jnp.where(lane == idx, _BIG, d2)
        idxs.append(idx)

    idx_ref[...] = jnp.concatenate(idxs, axis=1)


def _gather_body(table_hbm, idx_hbm, out_hbm, idx_v, rows_v, sem):
    wid = jax.lax.axis_index("s") * 2 + jax.lax.axis_index("c")
    base = wid * _RPW
    for c in range(_NCH):
        off = base + c * _CHUNK
        pltpu.sync_copy(idx_hbm.at[pl.ds(off, _CHUNK)], idx_v)
        pltpu.async_copy(table_hbm.at[idx_v], rows_v, sem).wait()
        pltpu.sync_copy(rows_v, out_hbm.at[pl.ds(off, _CHUNK)])


def _combine_body(g_ref, q_ref, out_ref):
    g = g_ref[...]
    q = q_ref[...]
    qx = q[:, 0:1]
    qy = q[:, 1:2]
    qz = q[:, 2:3]
    rs = []
    for k in range(_K):
        dx = g[:, k * _DP + _D:k * _DP + _D + 1] - qx
        dy = g[:, k * _DP + _D + 1:k * _DP + _D + 2] - qy
        dz = g[:, k * _DP + _D + 2:k * _DP + _D + 3] - qz
        dist = jnp.sqrt((dx * dx + dy * dy) + dz * dz)
        rs.append(1.0 / jnp.maximum(dist, 1e-10))
    norm = (rs[0] + rs[1]) + rs[2]
    out_ref[...] = (g[:, 0:_D] * (rs[0] / norm)
                    + g[:, _DP:_DP + _D] * (rs[1] / norm)) \
        + g[:, 2 * _DP:2 * _DP + _D] * (rs[2] / norm)


def kernel(f_ref_C, f_ref_F, f_cur_C, point_flow):
    f_ref_warp = pl.pallas_call(
        _warp_body,
        grid=(5,),
        in_specs=[
            pl.BlockSpec((_N1 // 5, 3), lambda i: (i, 0)),
            pl.BlockSpec((_N1 // 5, 3), lambda i: (i, 0)),
        ],
        out_specs=pl.BlockSpec((_N1 // 5, 3), lambda i: (i, 0)),
        out_shape=jax.ShapeDtypeStruct((_N1, 3), jnp.float32),
    )(f_ref_C, point_flow)

    srcT = jnp.pad(f_ref_warp, ((0, _N1P - _N1), (0, 0)),
                   constant_values=_PAD_COORD).T
    q_pad = jnp.pad(f_cur_C, ((0, _N2P - _N2), (0, 0)))
    table = jnp.pad(jnp.concatenate([f_ref_F, f_ref_warp], axis=1),
                    ((0, 0), (0, _DP - _D - 3)))

    knn_idx = pl.pallas_call(
        _knn_body,
        grid=(_N2P // _BQ,),
        in_specs=[
            pl.BlockSpec((_BQ, 3), lambda i: (i, 0)),
            pl.BlockSpec((3, _N1P), lambda i: (0, 0)),
        ],
        out_specs=pl.BlockSpec((_BQ, _K), lambda i: (i, 0)),
        out_shape=jax.ShapeDtypeStruct((_N2P, _K), jnp.int32),
    )(q_pad, srcT)

    idx_flat = knn_idx.reshape(_N2P * _K)
    mesh = plsc.VectorSubcoreMesh(core_axis_name="c", subcore_axis_name="s")
    grouped = pl.kernel(
        _gather_body,
        mesh=mesh,
        out_type=jax.ShapeDtypeStruct((_N2P * _K, _DP), jnp.float32),
        scratch_types=[
            pltpu.VMEM((_CHUNK,), jnp.int32),
            pltpu.VMEM((_CHUNK, _DP), jnp.float32),
            pltpu.SemaphoreType.DMA,
        ],
    )(table, idx_flat)

    out = pl.pallas_call(
        _combine_body,
        grid=(_N2P // _BC,),
        in_specs=[
            pl.BlockSpec((_BC, _K * _DP), lambda i: (i, 0)),
            pl.BlockSpec((_BC, 3), lambda i: (i, 0)),
        ],
        out_specs=pl.BlockSpec((_BC, _D), lambda i: (i, 0)),
        out_shape=jax.ShapeDtypeStruct((_N2P, _D), jnp.float32),
    )(grouped.reshape(_N2P, _K * _DP), q_pad)

    return out[:_N2], f_ref_warp
